# baseline (speedup 1.0000x reference)
import functools

import jax
import jax.numpy as jnp
from jax import lax
from jax.experimental import pallas as pl
from jax.experimental.pallas import tpu as pltpu
from jax.experimental.pallas import tpu_sc as plsc

B = 16384
NCOL = 17
NS = 8
VOCAB = 100000
EMB = 106
DENSE = NCOL - NS
DEMB = NS * EMB

NC = 2
NSUB = 16
NW = NC * NSUB
L = 16

CPW = 27
OUTCH = 4096


def _gather_sc(tabT, idxT):
    mesh = plsc.VectorSubcoreMesh(core_axis_name="c", subcore_axis_name="s")

    @functools.partial(
        pl.kernel,
        mesh=mesh,
        compiler_params=pltpu.CompilerParams(needs_layout_passes=False),
        out_type=jax.ShapeDtypeStruct((DEMB * B,), jnp.float32),
        scratch_types=[
            pltpu.VMEM((1, VOCAB), jnp.float32),
            pltpu.VMEM((B,), jnp.int32),
            pltpu.VMEM((2, OUTCH), jnp.float32),
            pltpu.SemaphoreType.DMA,
            pltpu.SemaphoreType.DMA,
        ],
    )
    def k(tabT_hbm, idxT_hbm, out_hbm, slab_v, idx_v, outb_v, sem0, sem1):
        wid = lax.axis_index("s") * NC + lax.axis_index("c")
        t = wid % NS
        c0 = (wid // NS) * CPW
        ncols = jnp.maximum(0, jnp.minimum(CPW, EMB - c0))
        zero16 = jnp.zeros((L,), jnp.int32)
        sems = [sem0, sem1]
        pltpu.sync_copy(idxT_hbm.at[pl.ds(t * B, B)], idx_v)

        def col_body(i, carry):
            j = (c0 + i) * NS + t
            pltpu.sync_copy(tabT_hbm.at[pl.ds(j, 1)], slab_v)
            for cc in range(B // OUTCH):
                buf = cc % 2
                if cc >= 2:
                    pltpu.make_async_copy(
                        outb_v.at[buf], out_hbm.at[pl.ds(0, OUTCH)],
                        sems[buf]).wait()
                ob = cc * OUTCH
                def chunk_body(q, carry2):
                    boff = q * (L * 8)
                    for u in range(8):
                        iv = idx_v[pl.ds(ob + boff + u * L, L)]
                        outb_v[buf, pl.ds(boff + u * L, L)] = (
                            plsc.load_gather(slab_v, [zero16, iv]))
                    return carry2
                lax.fori_loop(0, OUTCH // (L * 8), chunk_body, 0)
                pltpu.async_copy(outb_v.at[buf],
                                 out_hbm.at[pl.ds(j * B + ob, OUTCH)],
                                 sems[buf])
            for buf in range(2):
                pltpu.make_async_copy(
                    outb_v.at[buf], out_hbm.at[pl.ds(0, OUTCH)],
                    sems[buf]).wait()
            return carry

        lax.fori_loop(0, ncols, col_body, 0)

    return k(tabT, idxT)


BLK = 1024


def _rtne_bf16(x):
    u = lax.bitcast_convert_type(x, jnp.uint32)
    rb = (u >> 16) & jnp.uint32(1)
    u = (u + jnp.uint32(0x7FFF) + rb) & jnp.uint32(0xFFFF0000)
    return lax.bitcast_convert_type(u, jnp.float32).astype(jnp.bfloat16)


def _dotf(a_bf, b_bf):
    return jnp.dot(a_bf, b_bf, preferred_element_type=jnp.float32)


def _mlp_body(embt_ref, dense_ref, w1e_ref, w1d_ref, b1_ref, w2_ref, b2_ref,
              w3_ref, b3_ref, out_ref):
    et = _rtne_bf16(embt_ref[...])
    h = lax.dot_general(et, w1e_ref[...], (((0,), (0,)), ((), ())),
                        preferred_element_type=jnp.float32)
    h = h + _dotf(dense_ref[...], w1d_ref[...])
    h = jax.nn.relu(h + b1_ref[...])
    h = jax.nn.relu(_dotf(_rtne_bf16(h), w2_ref[...]) + b2_ref[...])
    o = _dotf(_rtne_bf16(h), w3_ref[...]) + b3_ref[...]
    out_ref[...] = 1.0 / (1.0 + jnp.exp(-o))


def _mlp_tc(embt, dense, W1e, W1d, b1, W2, b2, W3, b3):
    grid = (B // BLK,)
    return pl.pallas_call(
        _mlp_body,
        grid=grid,
        in_specs=[
            pl.BlockSpec((DEMB, BLK), lambda i: (0, i)),
            pl.BlockSpec((BLK, DENSE), lambda i: (i, 0)),
            pl.BlockSpec((DEMB, 128), lambda i: (0, 0)),
            pl.BlockSpec((DENSE, 128), lambda i: (0, 0)),
            pl.BlockSpec((1, 128), lambda i: (0, 0)),
            pl.BlockSpec((128, 128), lambda i: (0, 0)),
            pl.BlockSpec((1, 128), lambda i: (0, 0)),
            pl.BlockSpec((128, 1), lambda i: (0, 0)),
            pl.BlockSpec((1, 1), lambda i: (0, 0)),
        ],
        out_specs=pl.BlockSpec((BLK, 1), lambda i: (i, 0)),
        out_shape=jax.ShapeDtypeStruct((B, 1), jnp.float32),
    )(embt, dense, W1e, W1d, b1, W2, b2, W3, b3)


def kernel(feature, tables, W1, b1, W2, b2, W3, b3):
    f = feature.reshape(-1, NCOL)
    sparse = f[:, :NS].astype(jnp.int32)
    dense = f[:, NS:]
    tabT = jnp.transpose(tables, (2, 0, 1)).reshape(DEMB, VOCAB)
    idxT = sparse.T.reshape(-1)
    embt = _gather_sc(tabT, idxT).reshape(DEMB, B)
    bf = jnp.bfloat16
    W1e = W1[:DEMB].reshape(NS, EMB, 128).transpose(1, 0, 2).reshape(DEMB, 128)
    out = _mlp_tc(embt, dense.astype(bf), W1e.astype(bf), W1[DEMB:].astype(bf),
                  b1.reshape(1, 128), W2.astype(bf), b2.reshape(1, 128),
                  W3.astype(bf), b3.reshape(1, 1))
    return out.reshape(-1)


# speedup vs baseline: 1.0052x; 1.0052x over previous
import functools

import jax
import jax.numpy as jnp
from jax import lax
from jax.experimental import pallas as pl
from jax.experimental.pallas import tpu as pltpu
from jax.experimental.pallas import tpu_sc as plsc

B = 16384
NCOL = 17
NS = 8
VOCAB = 100000
EMB = 106
DENSE = NCOL - NS
DEMB = NS * EMB

NC = 2
NSUB = 16
NW = NC * NSUB
L = 16

CPW = 27
OUTCH = 4096


def _gather_sc(tabT, idxT):
    mesh = plsc.VectorSubcoreMesh(core_axis_name="c", subcore_axis_name="s")

    @functools.partial(
        pl.kernel,
        mesh=mesh,
        compiler_params=pltpu.CompilerParams(needs_layout_passes=False),
        out_type=jax.ShapeDtypeStruct((DEMB * B,), jnp.float32),
        scratch_types=[
            pltpu.VMEM((1, VOCAB), jnp.float32),
            pltpu.VMEM((B,), jnp.int32),
            pltpu.VMEM((2, OUTCH), jnp.float32),
            pltpu.SemaphoreType.DMA,
            pltpu.SemaphoreType.DMA,
            pltpu.SemaphoreType.DMA,
        ],
    )
    def k(tabT_hbm, idxT_hbm, out_hbm, slab_v, idx_v, outb_v, sem0, sem1,
          semS):
        wid = lax.axis_index("s") * NC + lax.axis_index("c")
        t = wid % NS
        c0 = (wid // NS) * CPW
        ncols = jnp.maximum(0, jnp.minimum(CPW, EMB - c0))
        zero16 = jnp.zeros((L,), jnp.int32)
        sems = [sem0, sem1]
        pltpu.sync_copy(idxT_hbm.at[pl.ds(t * B, B)], idx_v)

        def col_of(i):
            return jnp.minimum(c0 + i, EMB - 1) * NS + t

        def prefetch_slab(jrow):
            pltpu.async_copy(tabT_hbm.at[pl.ds(jrow, 1)], slab_v, semS)

        def wait_slab():
            pltpu.make_async_copy(
                tabT_hbm.at[pl.ds(0, 1)], slab_v, semS).wait()

        prefetch_slab(col_of(0))

        def col_body(i, carry):
            j = (c0 + i) * NS + t
            wait_slab()
            for cc in range(B // OUTCH):
                buf = cc % 2
                if cc >= 2:
                    pltpu.make_async_copy(
                        outb_v.at[buf], out_hbm.at[pl.ds(0, OUTCH)],
                        sems[buf]).wait()
                ob = cc * OUTCH
                def chunk_body(q, carry2):
                    boff = q * (L * 8)
                    for u in range(8):
                        iv = idx_v[pl.ds(ob + boff + u * L, L)]
                        outb_v[buf, pl.ds(boff + u * L, L)] = (
                            plsc.load_gather(slab_v, [zero16, iv]))
                    return carry2
                lax.fori_loop(0, OUTCH // (L * 8), chunk_body, 0)
                if cc == B // OUTCH - 1:
                    prefetch_slab(col_of(i + 1))
                pltpu.async_copy(outb_v.at[buf],
                                 out_hbm.at[pl.ds(j * B + ob, OUTCH)],
                                 sems[buf])
            for buf in range(2):
                pltpu.make_async_copy(
                    outb_v.at[buf], out_hbm.at[pl.ds(0, OUTCH)],
                    sems[buf]).wait()
            return carry

        lax.fori_loop(0, ncols, col_body, 0)
        wait_slab()

    return k(tabT, idxT)


BLK = 1024


def _rtne_bf16(x):
    u = lax.bitcast_convert_type(x, jnp.uint32)
    rb = (u >> 16) & jnp.uint32(1)
    u = (u + jnp.uint32(0x7FFF) + rb) & jnp.uint32(0xFFFF0000)
    return lax.bitcast_convert_type(u, jnp.float32).astype(jnp.bfloat16)


def _dotf(a_bf, b_bf):
    return jnp.dot(a_bf, b_bf, preferred_element_type=jnp.float32)


def _mlp_body(embt_ref, dense_ref, w1e_ref, w1d_ref, b1_ref, w2_ref, b2_ref,
              w3_ref, b3_ref, out_ref):
    et = _rtne_bf16(embt_ref[...])
    h = lax.dot_general(et, w1e_ref[...], (((0,), (0,)), ((), ())),
                        preferred_element_type=jnp.float32)
    h = h + _dotf(dense_ref[...], w1d_ref[...])
    h = jax.nn.relu(h + b1_ref[...])
    h = jax.nn.relu(_dotf(_rtne_bf16(h), w2_ref[...]) + b2_ref[...])
    o = _dotf(_rtne_bf16(h), w3_ref[...]) + b3_ref[...]
    out_ref[...] = 1.0 / (1.0 + jnp.exp(-o))


def _mlp_tc(embt, dense, W1e, W1d, b1, W2, b2, W3, b3):
    grid = (B // BLK,)
    return pl.pallas_call(
        _mlp_body,
        grid=grid,
        in_specs=[
            pl.BlockSpec((DEMB, BLK), lambda i: (0, i)),
            pl.BlockSpec((BLK, DENSE), lambda i: (i, 0)),
            pl.BlockSpec((DEMB, 128), lambda i: (0, 0)),
            pl.BlockSpec((DENSE, 128), lambda i: (0, 0)),
            pl.BlockSpec((1, 128), lambda i: (0, 0)),
            pl.BlockSpec((128, 128), lambda i: (0, 0)),
            pl.BlockSpec((1, 128), lambda i: (0, 0)),
            pl.BlockSpec((128, 1), lambda i: (0, 0)),
            pl.BlockSpec((1, 1), lambda i: (0, 0)),
        ],
        out_specs=pl.BlockSpec((BLK, 1), lambda i: (i, 0)),
        out_shape=jax.ShapeDtypeStruct((B, 1), jnp.float32),
    )(embt, dense, W1e, W1d, b1, W2, b2, W3, b3)


def kernel(feature, tables, W1, b1, W2, b2, W3, b3):
    f = feature.reshape(-1, NCOL)
    sparse = f[:, :NS].astype(jnp.int32)
    dense = f[:, NS:]
    tabT = jnp.transpose(tables, (2, 0, 1)).reshape(DEMB, VOCAB)
    idxT = sparse.T.reshape(-1)
    embt = _gather_sc(tabT, idxT).reshape(DEMB, B)
    bf = jnp.bfloat16
    W1e = W1[:DEMB].reshape(NS, EMB, 128).transpose(1, 0, 2).reshape(DEMB, 128)
    out = _mlp_tc(embt, dense.astype(bf), W1e.astype(bf), W1[DEMB:].astype(bf),
                  b1.reshape(1, 128), W2.astype(bf), b2.reshape(1, 128),
                  W3.astype(bf), b3.reshape(1, 1))
    return out.reshape(-1)
